# baseline (speedup 1.0000x reference)
import functools

import jax
import jax.numpy as jnp
import numpy as np
from jax import lax
from jax.experimental import pallas as pl
from jax.experimental.pallas import tpu as pltpu
from jax.experimental.pallas import tpu_sc as plsc

EMBED = 256
HEADS = 8
E = 32
LVL = 4
PT = 4
CAM = 6
QTOT = 1600
LEVEL_SHAPES = ((64, 176), (32, 88), (16, 44), (8, 22))
VALUE_NUM = sum(h * w for h, w in LEVEL_SHAPES)
ROWS_PER_PIX = CAM * HEADS
TABLE_ROWS = VALUE_NUM * ROWS_PER_PIX

QB = 200
NQB = QTOT // QB
NCORNER = 4
LP = LVL * PT
SAMP = HEADS * LP
SBLK = NCORNER * SAMP

NC = 2
NS = 16
NW = NC * NS
ITEMS = CAM * QTOT
ITEMS_PER_W = ITEMS // NW


def _lane_consts():
    lane = np.arange(SAMP)
    l_idx = (lane // PT) % LVL
    h_idx = lane // LP
    Hs = np.array([h for h, _ in LEVEL_SHAPES])
    Ws = np.array([w for _, w in LEVEL_SHAPES])
    starts = np.concatenate([[0], np.cumsum([h * w for h, w in LEVEL_SHAPES])])[:LVL]
    sx = Ws[l_idx].astype(np.float32)[None]
    sy = Hs[l_idx].astype(np.float32)[None]
    start_v = starts[l_idx].astype(np.int32)[None]
    wl_i = Ws[l_idx].astype(np.int32)[None]
    a_lane = ((h_idx // 4) * VALUE_NUM * 4 + (h_idx % 4)).astype(np.int32)[None]
    B = np.zeros((SAMP, HEADS), np.float32)
    B[lane, h_idx] = 1.0
    return sx, sy, start_v, wl_i, a_lane, B, B.T.copy()


PB0 = 880


def _stage0_body(v_ref, out_ref):
    for c in range(CAM):
        for t in range(2):
            out_ref[c * 2 + t] = v_ref[0, :, c, t * 128:(t + 1) * 128].astype(jnp.bfloat16)


def _stage0(value):
    return pl.pallas_call(
        _stage0_body,
        grid=(VALUE_NUM // PB0,),
        in_specs=[pl.BlockSpec((1, PB0, CAM, EMBED), lambda pb: (0, pb, 0, 0))],
        out_specs=pl.BlockSpec((2 * CAM, PB0, 128), lambda pb: (0, pb, 0)),
        out_shape=jax.ShapeDtypeStruct((2 * CAM, VALUE_NUM, 128), jnp.bfloat16),
    )(value)


def _stage1_body(q_ref, wox_ref, woy_ref, box_ref, boy_ref, ww_ref, bw_ref,
                 B_ref, BT_ref, sx_ref, sy_ref, start_ref, wli_ref, alane_ref,
                 pt_ref, idx_ref, cw_ref):
    c = pl.program_id(0)
    q = q_ref[...]
    offx = jnp.dot(q, wox_ref[...], preferred_element_type=jnp.float32) + box_ref[...]
    offy = jnp.dot(q, woy_ref[...], preferred_element_type=jnp.float32) + boy_ref[...]
    logit = jnp.dot(q, ww_ref[...], preferred_element_type=jnp.float32) + bw_ref[...]
    m = jnp.max(logit, axis=-1, keepdims=True)
    e = jnp.exp(logit - m)
    s = jnp.dot(e, B_ref[...], preferred_element_type=jnp.float32)
    r = jnp.dot(1.0 / s, BT_ref[...], preferred_element_type=jnp.float32)
    w = e * r

    sx = sx_ref[...]
    sy = sy_ref[...]
    px = pt_ref[0, :, 0:1]
    py = pt_ref[0, :, 1:2]
    x = px * sx + offx - 0.5
    y = py * sy + offy - 0.5
    x0 = jnp.floor(x)
    y0 = jnp.floor(y)
    fx1 = x - x0
    fx0 = 1.0 - fx1
    fy1 = y - y0
    fy0 = 1.0 - fy1
    wmax = sx - 1.0
    hmax = sy - 1.0
    vx0 = ((x0 >= 0.0) & (x0 <= wmax)).astype(jnp.float32)
    vx1 = ((x0 + 1.0 >= 0.0) & (x0 + 1.0 <= wmax)).astype(jnp.float32)
    vy0 = ((y0 >= 0.0) & (y0 <= hmax)).astype(jnp.float32)
    vy1 = ((y0 + 1.0 >= 0.0) & (y0 + 1.0 <= hmax)).astype(jnp.float32)
    ix0 = jnp.clip(x0, 0.0, wmax).astype(jnp.int32)
    ix1 = jnp.clip(x0 + 1.0, 0.0, wmax).astype(jnp.int32)
    iy0 = jnp.clip(y0, 0.0, hmax).astype(jnp.int32)
    iy1 = jnp.clip(y0 + 1.0, 0.0, hmax).astype(jnp.int32)

    start_v = start_ref[...]
    wl_i = wli_ref[...]
    rowoff = c * (2 * VALUE_NUM * 4) + alane_ref[...]
    corners = ((ix0, iy0, fx0 * fy0, vx0 * vy0),
               (ix1, iy0, fx1 * fy0, vx1 * vy0),
               (ix0, iy1, fx0 * fy1, vx0 * vy1),
               (ix1, iy1, fx1 * fy1, vx1 * vy1))
    for cn, (ix, iy, fw, valid) in enumerate(corners):
        pix = start_v + iy * wl_i + ix
        row = pix * 4 + rowoff
        idx_ref[0, cn] = row
        cw_ref[0, cn] = w * fw * valid


def _stage1(query2, point3, W_off, b_off, W_w, b_w):
    sx, sy, start_v, wl_i, a_lane, B, BT = _lane_consts()
    wox = W_off[:, 0::2]
    woy = W_off[:, 1::2]
    box = b_off[0::2][None]
    boy = b_off[1::2][None]
    bw = b_w[None]
    full = lambda c, qb: (0, 0)
    grid = (CAM, NQB)
    out_shapes = (jax.ShapeDtypeStruct((CAM, NCORNER, QTOT, SAMP), jnp.int32),
                  jax.ShapeDtypeStruct((CAM, NCORNER, QTOT, SAMP), jnp.float32))
    return pl.pallas_call(
        _stage1_body,
        grid=grid,
        in_specs=[
            pl.BlockSpec((QB, EMBED), lambda c, qb: (qb, 0)),
            pl.BlockSpec((EMBED, SAMP), full),
            pl.BlockSpec((EMBED, SAMP), full),
            pl.BlockSpec((1, SAMP), full),
            pl.BlockSpec((1, SAMP), full),
            pl.BlockSpec((EMBED, SAMP), full),
            pl.BlockSpec((1, SAMP), full),
            pl.BlockSpec((SAMP, HEADS), full),
            pl.BlockSpec((HEADS, SAMP), full),
            pl.BlockSpec((1, SAMP), full),
            pl.BlockSpec((1, SAMP), full),
            pl.BlockSpec((1, SAMP), full),
            pl.BlockSpec((1, SAMP), full),
            pl.BlockSpec((1, SAMP), full),
            pl.BlockSpec((1, QB, 2), lambda c, qb: (c, qb, 0)),
        ],
        out_specs=(pl.BlockSpec((1, NCORNER, QB, SAMP), lambda c, qb: (c, 0, qb, 0)),
                   pl.BlockSpec((1, NCORNER, QB, SAMP), lambda c, qb: (c, 0, qb, 0))),
        out_shape=out_shapes,
    )(query2, wox, woy, box, boy, W_w, bw, B, BT,
      jnp.asarray(sx), jnp.asarray(sy), jnp.asarray(start_v),
      jnp.asarray(wl_i), jnp.asarray(a_lane), point3)


def _sc_body(table_h, idx_h, cw_h, out_h, idx_v, cw_v, rows_v, out_v,
             sem_i, sem_g, sem_o):
    wid = lax.axis_index("s") * NC + lax.axis_index("c")
    base = wid * ITEMS_PER_W

    def start_idx(i, b):
        ci = lax.div(base + i, QTOT)
        qi = base + i - ci * QTOT
        for j in range(NCORNER):
            pltpu.async_copy(idx_h.at[ci, j, qi], idx_v.at[b, j], sem_i)

    def start_cw(i, b):
        ci = lax.div(base + i, QTOT)
        qi = base + i - ci * QTOT
        for j in range(NCORNER):
            pltpu.async_copy(cw_h.at[ci, j, qi], cw_v.at[b, j], sem_i)

    def wait_sem(src, dst, sem):
        pltpu.make_async_copy(src, dst, sem).wait()

    def wait_idx(b):
        for j in range(NCORNER):
            wait_sem(idx_h.at[0, 0, 0], idx_v.at[b, j], sem_i)

    def wait_cw(b):
        for j in range(NCORNER):
            wait_sem(cw_h.at[0, 0, 0], cw_v.at[b, j], sem_i)

    def issue_gathers(b):
        for j in range(NCORNER):
            pltpu.async_copy(table_h.at[idx_v.at[b, j]], rows_v.at[b, j], sem_g)

    def drain_gathers(b):
        for j in range(NCORNER):
            wait_sem(table_h.at[idx_v.at[b, j]], rows_v.at[b, j], sem_g)

    start_idx(0, 0)
    start_cw(0, 0)
    wait_idx(0)
    wait_cw(0)
    issue_gathers(0)
    start_idx(1, 1)
    start_cw(1, 1)

    @pl.loop(0, ITEMS_PER_W, step=2)
    def _(k):
        for b in (0, 1):
            i = k + b
            nb = 1 - b
            drain_gathers(b)

            @pl.when(i + 1 < ITEMS_PER_W)
            def _():
                wait_idx(nb)
                wait_cw(nb)
                issue_gathers(nb)

            @pl.when(i + 2 < ITEMS_PER_W)
            def _():
                start_idx(i + 2, b)

            @pl.when(i >= 2)
            def _():
                wait_sem(out_v.at[b], out_h.at[base], sem_o)

            for h in range(HEADS):
                def cbody(cn, carry2, h=h, b=b):
                    a0, a1 = carry2
                    wv = cw_v[b, cn, pl.ds(h * LP, LP)]
                    for jj in range(LP):
                        lane = h * LP + jj
                        wgt = wv[jj]
                        r = rows_v[b, cn, lane, pl.ds(0, E)]
                        ev, od = plsc.unpack(r, format=plsc.PackFormat.INTERLEAVED)
                        a0 = a0 + wgt * ev
                        a1 = a1 + wgt * od
                    return a0, a1
                a0, a1 = lax.fori_loop(0, NCORNER, cbody,
                                       (jnp.zeros((16,), jnp.float32),
                                        jnp.zeros((16,), jnp.float32)))
                out_v[b, pl.ds(h * E, 16)] = a0
                out_v[b, pl.ds(h * E + 16, 16)] = a1
            pltpu.async_copy(out_v.at[b], out_h.at[base + i], sem_o)

            @pl.when(i + 2 < ITEMS_PER_W)
            def _():
                start_cw(i + 2, b)

    wait_sem(out_v.at[0], out_h.at[base], sem_o)
    wait_sem(out_v.at[1], out_h.at[base], sem_o)


def _stage2(table4, idx4, cw4):
    mesh = plsc.VectorSubcoreMesh(core_axis_name="c", subcore_axis_name="s",
                                  num_cores=NC, num_subcores=NS)
    fn = pl.kernel(
        _sc_body,
        out_type=jax.ShapeDtypeStruct((ITEMS, HEADS * E), jnp.float32),
        mesh=mesh,
        scratch_types=[
            pltpu.VMEM((2, NCORNER, SAMP), jnp.int32),
            pltpu.VMEM((2, NCORNER, SAMP), jnp.float32),
            pltpu.VMEM((2, NCORNER, SAMP, E), jnp.bfloat16),
            pltpu.VMEM((2, HEADS * E), jnp.float32),
            pltpu.SemaphoreType.DMA,
            pltpu.SemaphoreType.DMA,
            pltpu.SemaphoreType.DMA,
        ],
        compiler_params=pltpu.CompilerParams(use_tc_tiling_on_sc=False,
                                             needs_layout_passes=False),
    )
    return fn(table4, idx4, cw4)


def _stage3_body(samp_ref, mask_ref, wp_ref, bp_ref, out_ref):
    m = mask_ref[...]
    s = samp_ref[...]
    slots = s[0] * m[:, 0:1]
    for c in range(1, CAM):
        slots = slots + s[c] * m[:, c:c + 1]
    cnt = jnp.maximum(jnp.sum(m, axis=1, keepdims=True), 1.0)
    out_ref[...] = (jnp.dot(slots / cnt, wp_ref[...],
                            preferred_element_type=jnp.float32) + bp_ref[...])


def _stage3(samp3, maskT, W_proj, b_proj):
    return pl.pallas_call(
        _stage3_body,
        grid=(NQB,),
        in_specs=[
            pl.BlockSpec((CAM, QB, EMBED), lambda qb: (0, qb, 0)),
            pl.BlockSpec((QB, CAM), lambda qb: (qb, 0)),
            pl.BlockSpec((EMBED, EMBED), lambda qb: (0, 0)),
            pl.BlockSpec((1, EMBED), lambda qb: (0, 0)),
        ],
        out_specs=pl.BlockSpec((QB, EMBED), lambda qb: (qb, 0)),
        out_shape=jax.ShapeDtypeStruct((QTOT, EMBED), jnp.float32),
    )(samp3, maskT, W_proj, b_proj[None])


def kernel(query, value, point, valid, shape, W_off, b_off, W_w, b_w, W_proj, b_proj):
    query2 = query[0]
    point3 = point[:, 0]
    idx, cw = _stage1(query2, point3, W_off, b_off, W_w, b_w)
    table4 = _stage0(value)
    samp = _stage2(table4.reshape(TABLE_ROWS, E), idx, cw)
    samp3 = samp.reshape(CAM, QTOT, EMBED)
    maskT = jnp.transpose(valid[:, 0]).astype(jnp.float32)
    k = np.arange(EMBED)
    r = k % E
    orig = (k // E) * E + np.where(r < 16, 2 * r, 2 * (r - 16) + 1)
    out = _stage3(samp3, maskT, W_proj[orig], b_proj)
    return out[None]


# speedup vs baseline: 1.2361x; 1.2361x over previous
import functools

import jax
import jax.numpy as jnp
import numpy as np
from jax import lax
from jax.experimental import pallas as pl
from jax.experimental.pallas import tpu as pltpu
from jax.experimental.pallas import tpu_sc as plsc

EMBED = 256
HEADS = 8
E = 32
LVL = 4
PT = 4
CAM = 6
QTOT = 1600
LEVEL_SHAPES = ((64, 176), (32, 88), (16, 44), (8, 22))
VALUE_NUM = sum(h * w for h, w in LEVEL_SHAPES)
ROWS_PER_PIX = CAM * HEADS
TABLE_ROWS = VALUE_NUM * ROWS_PER_PIX

QB = 200
NQB = QTOT // QB
NCORNER = 4
LP = LVL * PT
SAMP = HEADS * LP
SBLK = NCORNER * SAMP

NC = 2
NS = 16
NW = NC * NS
ITEMS = CAM * QTOT
ITEMS_PER_W = ITEMS // NW


def _lane_consts():
    lane = np.arange(SAMP)
    l_idx = (lane // PT) % LVL
    h_idx = lane // LP
    Hs = np.array([h for h, _ in LEVEL_SHAPES])
    Ws = np.array([w for _, w in LEVEL_SHAPES])
    starts = np.concatenate([[0], np.cumsum([h * w for h, w in LEVEL_SHAPES])])[:LVL]
    sx = Ws[l_idx].astype(np.float32)[None]
    sy = Hs[l_idx].astype(np.float32)[None]
    start_v = starts[l_idx].astype(np.int32)[None]
    wl_i = Ws[l_idx].astype(np.int32)[None]
    a_lane = ((h_idx // 4) * VALUE_NUM * 4 + (h_idx % 4)).astype(np.int32)[None]
    B = np.zeros((SAMP, HEADS), np.float32)
    B[lane, h_idx] = 1.0
    return sx, sy, start_v, wl_i, a_lane, B, B.T.copy()


PB0 = 880


def _stage0_body(v_ref, out_ref):
    for c in range(CAM):
        for t in range(2):
            out_ref[c * 2 + t] = v_ref[0, :, c, t * 128:(t + 1) * 128]


def _stage0(value):
    return pl.pallas_call(
        _stage0_body,
        grid=(VALUE_NUM // PB0,),
        in_specs=[pl.BlockSpec((1, PB0, CAM, EMBED), lambda pb: (0, pb, 0, 0))],
        out_specs=pl.BlockSpec((2 * CAM, PB0, 128), lambda pb: (0, pb, 0)),
        out_shape=jax.ShapeDtypeStruct((2 * CAM, VALUE_NUM, 128), jnp.float32),
    )(value)


def _stage1_body(q_ref, wox_ref, woy_ref, box_ref, boy_ref, ww_ref, bw_ref,
                 B_ref, BT_ref, sx_ref, sy_ref, start_ref, wli_ref, alane_ref,
                 pt_ref, idx_ref, cw_ref):
    c = pl.program_id(0)
    q = q_ref[...]
    offx = jnp.dot(q, wox_ref[...], preferred_element_type=jnp.float32) + box_ref[...]
    offy = jnp.dot(q, woy_ref[...], preferred_element_type=jnp.float32) + boy_ref[...]
    logit = jnp.dot(q, ww_ref[...], preferred_element_type=jnp.float32) + bw_ref[...]
    m = jnp.max(logit, axis=-1, keepdims=True)
    e = jnp.exp(logit - m)
    s = jnp.dot(e, B_ref[...], preferred_element_type=jnp.float32)
    r = jnp.dot(1.0 / s, BT_ref[...], preferred_element_type=jnp.float32)
    w = e * r

    sx = sx_ref[...]
    sy = sy_ref[...]
    px = pt_ref[0, :, 0:1]
    py = pt_ref[0, :, 1:2]
    x = px * sx + offx - 0.5
    y = py * sy + offy - 0.5
    x0 = jnp.floor(x)
    y0 = jnp.floor(y)
    fx1 = x - x0
    fx0 = 1.0 - fx1
    fy1 = y - y0
    fy0 = 1.0 - fy1
    wmax = sx - 1.0
    hmax = sy - 1.0
    vx0 = ((x0 >= 0.0) & (x0 <= wmax)).astype(jnp.float32)
    vx1 = ((x0 + 1.0 >= 0.0) & (x0 + 1.0 <= wmax)).astype(jnp.float32)
    vy0 = ((y0 >= 0.0) & (y0 <= hmax)).astype(jnp.float32)
    vy1 = ((y0 + 1.0 >= 0.0) & (y0 + 1.0 <= hmax)).astype(jnp.float32)
    ix0 = jnp.clip(x0, 0.0, wmax).astype(jnp.int32)
    ix1 = jnp.clip(x0 + 1.0, 0.0, wmax).astype(jnp.int32)
    iy0 = jnp.clip(y0, 0.0, hmax).astype(jnp.int32)
    iy1 = jnp.clip(y0 + 1.0, 0.0, hmax).astype(jnp.int32)

    start_v = start_ref[...]
    wl_i = wli_ref[...]
    rowoff = c * (2 * VALUE_NUM * 4) + alane_ref[...]
    corners = ((ix0, iy0, fx0 * fy0, vx0 * vy0),
               (ix1, iy0, fx1 * fy0, vx1 * vy0),
               (ix0, iy1, fx0 * fy1, vx0 * vy1),
               (ix1, iy1, fx1 * fy1, vx1 * vy1))
    for cn, (ix, iy, fw, valid) in enumerate(corners):
        pix = start_v + iy * wl_i + ix
        row = pix * 4 + rowoff
        idx_ref[0, cn] = row
        cw_ref[0, cn] = w * fw * valid


def _stage1(query2, point3, W_off, b_off, W_w, b_w):
    sx, sy, start_v, wl_i, a_lane, B, BT = _lane_consts()
    wox = W_off[:, 0::2]
    woy = W_off[:, 1::2]
    box = b_off[0::2][None]
    boy = b_off[1::2][None]
    bw = b_w[None]
    full = lambda c, qb: (0, 0)
    grid = (CAM, NQB)
    out_shapes = (jax.ShapeDtypeStruct((CAM, NCORNER, QTOT, SAMP), jnp.int32),
                  jax.ShapeDtypeStruct((CAM, NCORNER, QTOT, SAMP), jnp.float32))
    return pl.pallas_call(
        _stage1_body,
        grid=grid,
        in_specs=[
            pl.BlockSpec((QB, EMBED), lambda c, qb: (qb, 0)),
            pl.BlockSpec((EMBED, SAMP), full),
            pl.BlockSpec((EMBED, SAMP), full),
            pl.BlockSpec((1, SAMP), full),
            pl.BlockSpec((1, SAMP), full),
            pl.BlockSpec((EMBED, SAMP), full),
            pl.BlockSpec((1, SAMP), full),
            pl.BlockSpec((SAMP, HEADS), full),
            pl.BlockSpec((HEADS, SAMP), full),
            pl.BlockSpec((1, SAMP), full),
            pl.BlockSpec((1, SAMP), full),
            pl.BlockSpec((1, SAMP), full),
            pl.BlockSpec((1, SAMP), full),
            pl.BlockSpec((1, SAMP), full),
            pl.BlockSpec((1, QB, 2), lambda c, qb: (c, qb, 0)),
        ],
        out_specs=(pl.BlockSpec((1, NCORNER, QB, SAMP), lambda c, qb: (c, 0, qb, 0)),
                   pl.BlockSpec((1, NCORNER, QB, SAMP), lambda c, qb: (c, 0, qb, 0))),
        out_shape=out_shapes,
    )(query2, wox, woy, box, boy, W_w, bw, B, BT,
      jnp.asarray(sx), jnp.asarray(sy), jnp.asarray(start_v),
      jnp.asarray(wl_i), jnp.asarray(a_lane), point3)


def _sc_body(table_h, idx_h, cw_h, out_h, idx_v, cw_v, rows_v, out_v,
             sem_i, sem_g, sem_o):
    wid = lax.axis_index("s") * NC + lax.axis_index("c")
    base = wid * ITEMS_PER_W

    def start_idx(i, b):
        ci = lax.div(base + i, QTOT)
        qi = base + i - ci * QTOT
        for j in range(NCORNER):
            pltpu.async_copy(idx_h.at[ci, j, qi], idx_v.at[b, j], sem_i)

    def start_cw(i, b):
        ci = lax.div(base + i, QTOT)
        qi = base + i - ci * QTOT
        for j in range(NCORNER):
            pltpu.async_copy(cw_h.at[ci, j, qi], cw_v.at[b, j], sem_i)

    def wait_sem(src, dst, sem):
        pltpu.make_async_copy(src, dst, sem).wait()

    def wait_idx(b):
        for j in range(NCORNER):
            wait_sem(idx_h.at[0, 0, 0], idx_v.at[b, j], sem_i)

    def wait_cw(b):
        for j in range(NCORNER):
            wait_sem(cw_h.at[0, 0, 0], cw_v.at[b, j], sem_i)

    def issue_gathers(b):
        for j in range(NCORNER):
            pltpu.async_copy(table_h.at[idx_v.at[b, j]], rows_v.at[b, j], sem_g)

    def drain_gathers(b):
        for j in range(NCORNER):
            wait_sem(table_h.at[idx_v.at[b, j]], rows_v.at[b, j], sem_g)

    start_idx(0, 0)
    start_cw(0, 0)
    start_idx(1, 1)
    start_cw(1, 1)
    wait_idx(0)
    wait_cw(0)
    issue_gathers(0)
    wait_idx(1)
    wait_cw(1)
    issue_gathers(1)
    start_idx(2, 2)
    start_cw(2, 2)

    @pl.loop(0, ITEMS_PER_W, step=3)
    def _(k):
        for b in (0, 1, 2):
            i = k + b
            nb = (b + 2) % 3
            drain_gathers(b)

            @pl.when(i + 2 < ITEMS_PER_W)
            def _():
                wait_idx(nb)
                wait_cw(nb)
                issue_gathers(nb)

            @pl.when(i + 3 < ITEMS_PER_W)
            def _():
                start_idx(i + 3, b)

            @pl.when(i >= 3)
            def _():
                wait_sem(out_v.at[b], out_h.at[base], sem_o)

            for h in range(HEADS):
                def cbody(cn, carry2, h=h, b=b):
                    a0, a1 = carry2
                    wv = cw_v[b, cn, pl.ds(h * LP, LP)]
                    for jj in range(LP):
                        lane = h * LP + jj
                        wgt = wv[jj]
                        a0 = a0 + wgt * rows_v[b, cn, lane, pl.ds(0, 16)]
                        a1 = a1 + wgt * rows_v[b, cn, lane, pl.ds(16, 16)]
                    return a0, a1
                a0, a1 = lax.fori_loop(0, NCORNER, cbody,
                                       (jnp.zeros((16,), jnp.float32),
                                        jnp.zeros((16,), jnp.float32)))
                out_v[b, pl.ds(h * E, 16)] = a0
                out_v[b, pl.ds(h * E + 16, 16)] = a1
            pltpu.async_copy(out_v.at[b], out_h.at[base + i], sem_o)

            @pl.when(i + 3 < ITEMS_PER_W)
            def _():
                start_cw(i + 3, b)

    wait_sem(out_v.at[0], out_h.at[base], sem_o)
    wait_sem(out_v.at[1], out_h.at[base], sem_o)
    wait_sem(out_v.at[2], out_h.at[base], sem_o)


def _stage2(table4, idx4, cw4):
    mesh = plsc.VectorSubcoreMesh(core_axis_name="c", subcore_axis_name="s",
                                  num_cores=NC, num_subcores=NS)
    fn = pl.kernel(
        _sc_body,
        out_type=jax.ShapeDtypeStruct((ITEMS, HEADS * E), jnp.float32),
        mesh=mesh,
        scratch_types=[
            pltpu.VMEM((3, NCORNER, SAMP), jnp.int32),
            pltpu.VMEM((3, NCORNER, SAMP), jnp.float32),
            pltpu.VMEM((3, NCORNER, SAMP, E), jnp.float32),
            pltpu.VMEM((3, HEADS * E), jnp.float32),
            pltpu.SemaphoreType.DMA,
            pltpu.SemaphoreType.DMA,
            pltpu.SemaphoreType.DMA,
        ],
        compiler_params=pltpu.CompilerParams(use_tc_tiling_on_sc=False),
    )
    return fn(table4, idx4, cw4)


def _stage3_body(samp_ref, mask_ref, wp_ref, bp_ref, out_ref):
    m = mask_ref[...]
    s = samp_ref[...]
    slots = s[0] * m[:, 0:1]
    for c in range(1, CAM):
        slots = slots + s[c] * m[:, c:c + 1]
    cnt = jnp.maximum(jnp.sum(m, axis=1, keepdims=True), 1.0)
    out_ref[...] = (jnp.dot(slots / cnt, wp_ref[...],
                            preferred_element_type=jnp.float32) + bp_ref[...])


def _stage3(samp3, maskT, W_proj, b_proj):
    return pl.pallas_call(
        _stage3_body,
        grid=(NQB,),
        in_specs=[
            pl.BlockSpec((CAM, QB, EMBED), lambda qb: (0, qb, 0)),
            pl.BlockSpec((QB, CAM), lambda qb: (qb, 0)),
            pl.BlockSpec((EMBED, EMBED), lambda qb: (0, 0)),
            pl.BlockSpec((1, EMBED), lambda qb: (0, 0)),
        ],
        out_specs=pl.BlockSpec((QB, EMBED), lambda qb: (qb, 0)),
        out_shape=jax.ShapeDtypeStruct((QTOT, EMBED), jnp.float32),
    )(samp3, maskT, W_proj, b_proj[None])


def kernel(query, value, point, valid, shape, W_off, b_off, W_w, b_w, W_proj, b_proj):
    query2 = query[0]
    point3 = point[:, 0]
    idx, cw = _stage1(query2, point3, W_off, b_off, W_w, b_w)
    table4 = _stage0(value)
    samp = _stage2(table4.reshape(TABLE_ROWS, E), idx, cw)
    samp3 = samp.reshape(CAM, QTOT, EMBED)
    maskT = jnp.transpose(valid[:, 0]).astype(jnp.float32)
    out = _stage3(samp3, maskT, W_proj, b_proj)
    return out[None]


# speedup vs baseline: 1.2795x; 1.0351x over previous
import functools

import jax
import jax.numpy as jnp
import numpy as np
from jax import lax
from jax.experimental import pallas as pl
from jax.experimental.pallas import tpu as pltpu
from jax.experimental.pallas import tpu_sc as plsc

EMBED = 256
HEADS = 8
E = 32
LVL = 4
PT = 4
CAM = 6
QTOT = 1600
LEVEL_SHAPES = ((64, 176), (32, 88), (16, 44), (8, 22))
VALUE_NUM = sum(h * w for h, w in LEVEL_SHAPES)
ROWS_PER_PIX = CAM * HEADS
TABLE_ROWS = VALUE_NUM * ROWS_PER_PIX

QB = 200
NQB = QTOT // QB
NCORNER = 4
LP = LVL * PT
SAMP = HEADS * LP
SBLK = NCORNER * SAMP

NC = 2
NS = 16
NW = NC * NS
ITEMS = CAM * QTOT
ITEMS_PER_W = ITEMS // NW


def _lane_consts():
    lane = np.arange(SAMP)
    l_idx = (lane // PT) % LVL
    h_idx = lane // LP
    Hs = np.array([h for h, _ in LEVEL_SHAPES])
    Ws = np.array([w for _, w in LEVEL_SHAPES])
    starts = np.concatenate([[0], np.cumsum([h * w for h, w in LEVEL_SHAPES])])[:LVL]
    sx = Ws[l_idx].astype(np.float32)[None]
    sy = Hs[l_idx].astype(np.float32)[None]
    start_v = starts[l_idx].astype(np.int32)[None]
    wl_i = Ws[l_idx].astype(np.int32)[None]
    a_lane = ((h_idx // 4) * VALUE_NUM * 4 + (h_idx % 4)).astype(np.int32)[None]
    B = np.zeros((SAMP, HEADS), np.float32)
    B[lane, h_idx] = 1.0
    return sx, sy, start_v, wl_i, a_lane, B, B.T.copy()


PB0 = 880


def _stage0_body(v_ref, out_ref):
    for c in range(CAM):
        for t in range(2):
            out_ref[c * 2 + t] = v_ref[0, :, c, t * 128:(t + 1) * 128]


def _stage0(value):
    return pl.pallas_call(
        _stage0_body,
        grid=(VALUE_NUM // PB0,),
        in_specs=[pl.BlockSpec((1, PB0, CAM, EMBED), lambda pb: (0, pb, 0, 0))],
        out_specs=pl.BlockSpec((2 * CAM, PB0, 128), lambda pb: (0, pb, 0)),
        out_shape=jax.ShapeDtypeStruct((2 * CAM, VALUE_NUM, 128), jnp.float32),
    )(value)


def _stage1_body(q_ref, wox_ref, woy_ref, box_ref, boy_ref, ww_ref, bw_ref,
                 B_ref, BT_ref, sx_ref, sy_ref, start_ref, wli_ref, alane_ref,
                 pt_ref, icw_ref):
    c = pl.program_id(0)
    q = q_ref[...]
    offx = jnp.dot(q, wox_ref[...], preferred_element_type=jnp.float32) + box_ref[...]
    offy = jnp.dot(q, woy_ref[...], preferred_element_type=jnp.float32) + boy_ref[...]
    logit = jnp.dot(q, ww_ref[...], preferred_element_type=jnp.float32) + bw_ref[...]
    m = jnp.max(logit, axis=-1, keepdims=True)
    e = jnp.exp(logit - m)
    s = jnp.dot(e, B_ref[...], preferred_element_type=jnp.float32)
    r = jnp.dot(1.0 / s, BT_ref[...], preferred_element_type=jnp.float32)
    w = e * r

    sx = sx_ref[...]
    sy = sy_ref[...]
    px = pt_ref[0, :, 0:1]
    py = pt_ref[0, :, 1:2]
    x = px * sx + offx - 0.5
    y = py * sy + offy - 0.5
    x0 = jnp.floor(x)
    y0 = jnp.floor(y)
    fx1 = x - x0
    fx0 = 1.0 - fx1
    fy1 = y - y0
    fy0 = 1.0 - fy1
    wmax = sx - 1.0
    hmax = sy - 1.0
    vx0 = ((x0 >= 0.0) & (x0 <= wmax)).astype(jnp.float32)
    vx1 = ((x0 + 1.0 >= 0.0) & (x0 + 1.0 <= wmax)).astype(jnp.float32)
    vy0 = ((y0 >= 0.0) & (y0 <= hmax)).astype(jnp.float32)
    vy1 = ((y0 + 1.0 >= 0.0) & (y0 + 1.0 <= hmax)).astype(jnp.float32)
    ix0 = jnp.clip(x0, 0.0, wmax).astype(jnp.int32)
    ix1 = jnp.clip(x0 + 1.0, 0.0, wmax).astype(jnp.int32)
    iy0 = jnp.clip(y0, 0.0, hmax).astype(jnp.int32)
    iy1 = jnp.clip(y0 + 1.0, 0.0, hmax).astype(jnp.int32)

    start_v = start_ref[...]
    wl_i = wli_ref[...]
    rowoff = c * (2 * VALUE_NUM * 4) + alane_ref[...]
    corners = ((ix0, iy0, fx0 * fy0, vx0 * vy0),
               (ix1, iy0, fx1 * fy0, vx1 * vy0),
               (ix0, iy1, fx0 * fy1, vx0 * vy1),
               (ix1, iy1, fx1 * fy1, vx1 * vy1))
    for cn, (ix, iy, fw, valid) in enumerate(corners):
        pix = start_v + iy * wl_i + ix
        row = pix * 4 + rowoff
        icw_ref[0, :, cn, :] = row
        icw_ref[0, :, NCORNER + cn, :] = lax.bitcast_convert_type(
            w * fw * valid, jnp.int32)


def _stage1(query2, point3, W_off, b_off, W_w, b_w):
    sx, sy, start_v, wl_i, a_lane, B, BT = _lane_consts()
    wox = W_off[:, 0::2]
    woy = W_off[:, 1::2]
    box = b_off[0::2][None]
    boy = b_off[1::2][None]
    bw = b_w[None]
    full = lambda c, qb: (0, 0)
    grid = (CAM, NQB)
    out_shapes = jax.ShapeDtypeStruct((CAM, QTOT, 2 * NCORNER, SAMP), jnp.int32)
    return pl.pallas_call(
        _stage1_body,
        grid=grid,
        in_specs=[
            pl.BlockSpec((QB, EMBED), lambda c, qb: (qb, 0)),
            pl.BlockSpec((EMBED, SAMP), full),
            pl.BlockSpec((EMBED, SAMP), full),
            pl.BlockSpec((1, SAMP), full),
            pl.BlockSpec((1, SAMP), full),
            pl.BlockSpec((EMBED, SAMP), full),
            pl.BlockSpec((1, SAMP), full),
            pl.BlockSpec((SAMP, HEADS), full),
            pl.BlockSpec((HEADS, SAMP), full),
            pl.BlockSpec((1, SAMP), full),
            pl.BlockSpec((1, SAMP), full),
            pl.BlockSpec((1, SAMP), full),
            pl.BlockSpec((1, SAMP), full),
            pl.BlockSpec((1, SAMP), full),
            pl.BlockSpec((1, QB, 2), lambda c, qb: (c, qb, 0)),
        ],
        out_specs=pl.BlockSpec((1, QB, 2 * NCORNER, SAMP), lambda c, qb: (c, qb, 0, 0)),
        out_shape=out_shapes,
    )(query2, wox, woy, box, boy, W_w, bw, B, BT,
      jnp.asarray(sx), jnp.asarray(sy), jnp.asarray(start_v),
      jnp.asarray(wl_i), jnp.asarray(a_lane), point3)


def _sc_body(table_h, icw_h, out_h, icw_v, rows_v, out_v,
             sem_i, sem_g, sem_o):
    wid = lax.axis_index("s") * NC + lax.axis_index("c")
    base = wid * ITEMS_PER_W

    def start_icw(i, b):
        ci = lax.div(base + i, QTOT)
        qi = base + i - ci * QTOT
        pltpu.async_copy(icw_h.at[ci, qi], icw_v.at[b], sem_i)

    def wait_sem(src, dst, sem):
        pltpu.make_async_copy(src, dst, sem).wait()

    def wait_icw(b):
        wait_sem(icw_h.at[0, 0], icw_v.at[b], sem_i)

    def issue_gathers(b):
        for j in range(NCORNER):
            pltpu.async_copy(table_h.at[icw_v.at[b, j]], rows_v.at[b, j], sem_g)

    def drain_gathers(b):
        for j in range(NCORNER):
            wait_sem(table_h.at[icw_v.at[b, j]], rows_v.at[b, j], sem_g)

    start_icw(0, 0)
    wait_icw(0)
    issue_gathers(0)
    start_icw(1, 1)

    @pl.loop(0, ITEMS_PER_W, step=2)
    def _(k):
        for b in (0, 1):
            i = k + b
            nb = 1 - b
            drain_gathers(b)

            @pl.when(i + 1 < ITEMS_PER_W)
            def _():
                wait_icw(nb)
                issue_gathers(nb)

            @pl.when(i >= 2)
            def _():
                wait_sem(out_v.at[b], out_h.at[base], sem_o)

            for h in range(HEADS):
                def cbody(cn, carry2, h=h, b=b):
                    a0, a1 = carry2
                    wv = lax.bitcast_convert_type(
                        icw_v[b, NCORNER + cn, pl.ds(h * LP, LP)], jnp.float32)
                    for jj in range(LP):
                        lane = h * LP + jj
                        wgt = wv[jj]
                        a0 = a0 + wgt * rows_v[b, cn, lane, pl.ds(0, 16)]
                        a1 = a1 + wgt * rows_v[b, cn, lane, pl.ds(16, 16)]
                    return a0, a1
                a0, a1 = lax.fori_loop(0, NCORNER, cbody,
                                       (jnp.zeros((16,), jnp.float32),
                                        jnp.zeros((16,), jnp.float32)))
                out_v[b, pl.ds(h * E, 16)] = a0
                out_v[b, pl.ds(h * E + 16, 16)] = a1
            pltpu.async_copy(out_v.at[b], out_h.at[base + i], sem_o)

            @pl.when(i + 2 < ITEMS_PER_W)
            def _():
                start_icw(i + 2, b)

    wait_sem(out_v.at[0], out_h.at[base], sem_o)
    wait_sem(out_v.at[1], out_h.at[base], sem_o)


def _stage2(table2, icw):
    mesh = plsc.VectorSubcoreMesh(core_axis_name="c", subcore_axis_name="s",
                                  num_cores=NC, num_subcores=NS)
    fn = pl.kernel(
        _sc_body,
        out_type=jax.ShapeDtypeStruct((ITEMS, HEADS * E), jnp.float32),
        mesh=mesh,
        scratch_types=[
            pltpu.VMEM((2, 2 * NCORNER, SAMP), jnp.int32),
            pltpu.VMEM((2, NCORNER, SAMP, E), jnp.float32),
            pltpu.VMEM((2, HEADS * E), jnp.float32),
            pltpu.SemaphoreType.DMA,
            pltpu.SemaphoreType.DMA,
            pltpu.SemaphoreType.DMA,
        ],
        compiler_params=pltpu.CompilerParams(use_tc_tiling_on_sc=False),
    )
    return fn(table2, icw)


def _stage3_body(samp_ref, mask_ref, wp_ref, bp_ref, out_ref):
    m = mask_ref[...]
    s = samp_ref[...]
    slots = s[0] * m[:, 0:1]
    for c in range(1, CAM):
        slots = slots + s[c] * m[:, c:c + 1]
    cnt = jnp.maximum(jnp.sum(m, axis=1, keepdims=True), 1.0)
    out_ref[...] = (jnp.dot(slots / cnt, wp_ref[...],
                            preferred_element_type=jnp.float32) + bp_ref[...])


def _stage3(samp3, maskT, W_proj, b_proj):
    return pl.pallas_call(
        _stage3_body,
        grid=(NQB,),
        in_specs=[
            pl.BlockSpec((CAM, QB, EMBED), lambda qb: (0, qb, 0)),
            pl.BlockSpec((QB, CAM), lambda qb: (qb, 0)),
            pl.BlockSpec((EMBED, EMBED), lambda qb: (0, 0)),
            pl.BlockSpec((1, EMBED), lambda qb: (0, 0)),
        ],
        out_specs=pl.BlockSpec((QB, EMBED), lambda qb: (qb, 0)),
        out_shape=jax.ShapeDtypeStruct((QTOT, EMBED), jnp.float32),
    )(samp3, maskT, W_proj, b_proj[None])


def kernel(query, value, point, valid, shape, W_off, b_off, W_w, b_w, W_proj, b_proj):
    query2 = query[0]
    point3 = point[:, 0]
    icw = _stage1(query2, point3, W_off, b_off, W_w, b_w)
    table4 = _stage0(value)
    samp = _stage2(table4.reshape(TABLE_ROWS, E), icw)
    samp3 = samp.reshape(CAM, QTOT, EMBED)
    maskT = jnp.transpose(valid[:, 0]).astype(jnp.float32)
    out = _stage3(samp3, maskT, W_proj, b_proj)
    return out[None]
